# baseline (speedup 1.0000x reference)
import jax
import jax.numpy as jnp
from jax import lax
from jax.experimental import pallas as pl
from jax.experimental.pallas import tpu as pltpu
from jax.experimental.pallas import tpu_sc as plsc

N = 10000
E = 320000
D = 128

NC = 2
NS = 16
NW = NC * NS
C = 80
NCH = 128
EP = NCH * C
EPAD = NW * EP

ROWS_PER_TILE = 624
TAIL0 = NS * ROWS_PER_TILE
TAIL = N - TAIL0


def _mm_block(x_ref, w_ref, o_ref):
    o_ref[...] = jnp.dot(x_ref[...], w_ref[...],
                         preferred_element_type=jnp.float32)


def _matmul(x, w):
    grid = 10
    bn = N // grid
    return pl.pallas_call(
        _mm_block,
        grid=(grid,),
        in_specs=[
            pl.BlockSpec((bn, D), lambda i: (i, 0)),
            pl.BlockSpec((D, D), lambda i: (0, 0)),
        ],
        out_specs=pl.BlockSpec((bn, D), lambda i: (i, 0)),
        out_shape=jax.ShapeDtypeStruct((N, D), jnp.float32),
    )(x, w)


def _combine_block(p_ref, o_ref):
    o_ref[...] = jnp.maximum(p_ref[0] + p_ref[1], 0.0)


def _combine(partials):
    grid = 10
    bn = N // grid
    return pl.pallas_call(
        _combine_block,
        grid=(grid,),
        in_specs=[pl.BlockSpec((NC, bn, D), lambda i: (0, i, 0))],
        out_specs=pl.BlockSpec((bn, D), lambda i: (i, 0)),
        out_shape=jax.ShapeDtypeStruct((N, D), jnp.float32),
    )(partials)


_DNUMS = lax.GatherDimensionNumbers(
    offset_dims=(), collapsed_slice_dims=(0,), start_index_map=(0,))


def _scale_chunk(rows_ref, w_ref, b):
    @plsc.parallel_loop(0, C // 16, unroll=2)
    def group(g):
        w16 = w_ref[b, pl.ds(g * 16, 16)]
        for i in range(16):
            wb = lax.gather(
                w16, jnp.full((16, 1), i, jnp.int32), _DNUMS,
                slice_sizes=(1,),
                mode=lax.GatherScatterMode.PROMISE_IN_BOUNDS)
            r = g * 16 + i
            for d in range(D // 16):
                sl = pl.ds(d * 16, 16)
                rows_ref[r, sl] = rows_ref[r, sl] * wb


def _sc_body(support_hbm, src_hbm, dst_hbm, w_hbm, out_hbm,
             idx3, w_v, rows0, rows1, rows2, rows3, acc,
             ssem0, ssem1, ssem2, ssem3, dsem0, dsem1, dsem2, dsem3,
             wsem0, wsem1, wsem2, wsem3,
             gsem0, gsem1, gsem2, gsem3, csem0, csem1, csem2, csem3):
    c = lax.axis_index("c")
    s = lax.axis_index("s")
    wid = s * NC + c
    base = wid * EP

    rows = (rows0, rows1, rows2, rows3)
    ssems = (ssem0, ssem1, ssem2, ssem3)
    dsems = (dsem0, dsem1, dsem2, dsem3)
    wsems = (wsem0, wsem1, wsem2, wsem3)
    gsems = (gsem0, gsem1, gsem2, gsem3)
    csems = (csem0, csem1, csem2, csem3)


    def idx_descs(t, b):
        off = base + t * C
        return (
            pltpu.make_async_copy(src_hbm.at[pl.ds(off, C)],
                                  idx3.at[0, b], ssems[b]),
            pltpu.make_async_copy(dst_hbm.at[pl.ds(off, C)],
                                  idx3.at[1, b], dsems[b]),
            pltpu.make_async_copy(w_hbm.at[pl.ds(off, C)],
                                  w_v.at[b], wsems[b]),
        )

    def gather_desc(b):
        return pltpu.make_async_copy(support_hbm.at[idx3.at[0, b]],
                                     rows[b], gsems[b])

    def scatter_start(b):
        pltpu.async_copy(rows[b], acc.at[idx3.at[2, b]], csems[b], add=True)

    def scatter_wait(b):
        pltpu.make_async_copy(rows[b], acc.at[idx3.at[2, b]],
                              csems[b]).wait()

    def snapshot_dst(b):
        for j in range(C // 16):
            sl = pl.ds(j * 16, 16)
            idx3[2, b, sl] = idx3[1, b, sl]

    zv = jnp.zeros((16,), jnp.float32)

    def zrow(r, carry):
        for d in range(D // 16):
            rows0[r, pl.ds(d * 16, 16)] = zv
        return carry

    lax.fori_loop(0, C, zrow, 0, unroll=4)
    row0 = s * ROWS_PER_TILE
    for j in range(ROWS_PER_TILE // C):
        pltpu.sync_copy(rows0, acc.at[pl.ds(row0 + j * C, C)])
    rem = ROWS_PER_TILE % C
    pltpu.sync_copy(rows0.at[pl.ds(0, rem)],
                    acc.at[pl.ds(row0 + ROWS_PER_TILE - rem, rem)])

    @pl.when(s == NS - 1)
    def _zero_tail():
        pltpu.sync_copy(rows0.at[pl.ds(0, TAIL)], acc.at[pl.ds(TAIL0, TAIL)])

    for t in (0, 1, 2, 3):
        for d in idx_descs(t, t):
            d.start()
    for t in (0, 1, 2):
        for d in idx_descs(t, t):
            d.wait()
        gather_desc(t).start()

    plsc.subcore_barrier()

    def step(i, carry):
        t0 = i * 4
        for b in (0, 1, 2, 3):
            t = t0 + b
            pb = (b + 3) % 4

            gather_desc(b).wait()
            _scale_chunk(rows[b], w_v, b)
            snapshot_dst(b)

            @pl.when(t >= 1)
            def _drain():
                scatter_wait(pb)

            scatter_start(b)

            @pl.when(t + 3 < NCH)
            def _gather_ahead():
                for d in idx_descs(t + 3, pb):
                    d.wait()
                gather_desc(pb).start()

            @pl.when(t + 4 < NCH)
            def _prefetch():
                for d in idx_descs(t + 4, b):
                    d.start()
        return carry

    lax.fori_loop(0, NCH // 4, step, 0, unroll=False)

    scatter_wait((NCH - 1) % 4)

    plsc.subcore_barrier()
    pltpu.sync_copy(acc.at[pl.ds(row0, ROWS_PER_TILE)],
                    out_hbm.at[c, pl.ds(row0, ROWS_PER_TILE)])

    @pl.when(s == NS - 1)
    def _read_tail():
        pltpu.sync_copy(acc.at[pl.ds(TAIL0, TAIL)],
                        out_hbm.at[c, pl.ds(TAIL0, TAIL)])


def _sc_spmm(support, src_p, dst_p, w_p):
    mesh = plsc.VectorSubcoreMesh(core_axis_name="c", subcore_axis_name="s")
    f = pl.kernel(
        _sc_body,
        out_type=jax.ShapeDtypeStruct((NC, N, D), jnp.float32),
        mesh=mesh,
        scratch_types=[
            pltpu.VMEM((3, 4, C), jnp.int32),
            pltpu.VMEM((4, C), jnp.float32),
            pltpu.VMEM((C, D), jnp.float32),
            pltpu.VMEM((C, D), jnp.float32),
            pltpu.VMEM((C, D), jnp.float32),
            pltpu.VMEM((C, D), jnp.float32),
            pltpu.VMEM_SHARED((N, D), jnp.float32),
        ] + [pltpu.SemaphoreType.DMA] * 20,
    )
    return f(support, src_p, dst_p, w_p)


def kernel(input, edge_index, edge_weight, W):
    src = edge_index[0].astype(jnp.int32)
    dst = edge_index[1].astype(jnp.int32)
    pad = EPAD - E
    spread = (jnp.arange(pad, dtype=jnp.int32) * 13) % N
    src_p = jnp.concatenate([src, spread])
    dst_p = jnp.concatenate([dst, spread])
    w_p = jnp.pad(edge_weight, (0, pad))
    support = _matmul(input, W)
    partials = _sc_spmm(support, src_p, dst_p, w_p)
    return _combine(partials)


# speedup vs baseline: 1.0563x; 1.0563x over previous
import jax
import jax.numpy as jnp
from jax import lax
from jax.experimental import pallas as pl
from jax.experimental.pallas import tpu as pltpu
from jax.experimental.pallas import tpu_sc as plsc

N = 10000
E = 320000
D = 128

NC = 2
NS = 16
NW = NC * NS
C = 112
NCH = 90
EP = NCH * C
EPAD = NW * EP

ROWS_PER_TILE = 624
TAIL0 = NS * ROWS_PER_TILE
TAIL = N - TAIL0


def _mm_block(x_ref, w_ref, o_ref):
    o_ref[...] = jnp.dot(x_ref[...], w_ref[...],
                         preferred_element_type=jnp.float32)


def _matmul(x, w):
    grid = 10
    bn = N // grid
    return pl.pallas_call(
        _mm_block,
        grid=(grid,),
        in_specs=[
            pl.BlockSpec((bn, D), lambda i: (i, 0)),
            pl.BlockSpec((D, D), lambda i: (0, 0)),
        ],
        out_specs=pl.BlockSpec((bn, D), lambda i: (i, 0)),
        out_shape=jax.ShapeDtypeStruct((N, D), jnp.float32),
    )(x, w)


def _combine_block(p_ref, o_ref):
    o_ref[...] = jnp.maximum(p_ref[0] + p_ref[1], 0.0)


def _combine(partials):
    grid = 10
    bn = N // grid
    return pl.pallas_call(
        _combine_block,
        grid=(grid,),
        in_specs=[pl.BlockSpec((NC, bn, D), lambda i: (0, i, 0))],
        out_specs=pl.BlockSpec((bn, D), lambda i: (i, 0)),
        out_shape=jax.ShapeDtypeStruct((N, D), jnp.float32),
    )(partials)


_DNUMS = lax.GatherDimensionNumbers(
    offset_dims=(), collapsed_slice_dims=(0,), start_index_map=(0,))


def _scale_chunk(rows_ref, w_ref, b):
    @plsc.parallel_loop(0, C // 16, unroll=2)
    def group(g):
        w16 = w_ref[b, pl.ds(g * 16, 16)]
        for i in range(16):
            wb = lax.gather(
                w16, jnp.full((16, 1), i, jnp.int32), _DNUMS,
                slice_sizes=(1,),
                mode=lax.GatherScatterMode.PROMISE_IN_BOUNDS)
            r = g * 16 + i
            for d in range(D // 16):
                sl = pl.ds(d * 16, 16)
                rows_ref[r, sl] = rows_ref[r, sl] * wb


def _sc_body(support_hbm, src_hbm, dst_hbm, w_hbm, out_hbm,
             idx3, w_v, rows0, rows1, rows2, acc,
             ssem0, ssem1, ssem2, dsem0, dsem1, dsem2,
             wsem0, wsem1, wsem2,
             gsem0, gsem1, gsem2, csem0, csem1, csem2, zsem):
    c = lax.axis_index("c")
    s = lax.axis_index("s")
    wid = s * NC + c
    base = wid * EP

    rows = (rows0, rows1, rows2)
    ssems = (ssem0, ssem1, ssem2)
    dsems = (dsem0, dsem1, dsem2)
    wsems = (wsem0, wsem1, wsem2)
    gsems = (gsem0, gsem1, gsem2)
    csems = (csem0, csem1, csem2)


    def idx_descs(t, b):
        off = base + t * C
        return (
            pltpu.make_async_copy(src_hbm.at[pl.ds(off, C)],
                                  idx3.at[0, b], ssems[b]),
            pltpu.make_async_copy(dst_hbm.at[pl.ds(off, C)],
                                  idx3.at[1, b], dsems[b]),
            pltpu.make_async_copy(w_hbm.at[pl.ds(off, C)],
                                  w_v.at[b], wsems[b]),
        )

    def gather_desc(b):
        return pltpu.make_async_copy(support_hbm.at[idx3.at[0, b]],
                                     rows[b], gsems[b])

    def scatter_start(b):
        pltpu.async_copy(rows[b], acc.at[idx3.at[2, b]], csems[b], add=True)

    def scatter_wait(b):
        pltpu.make_async_copy(rows[b], acc.at[idx3.at[2, b]],
                              csems[b]).wait()

    def snapshot_dst(b):
        for j in range(C // 16):
            sl = pl.ds(j * 16, 16)
            idx3[2, b, sl] = idx3[1, b, sl]

    zv = jnp.zeros((16,), jnp.float32)

    def zrow(r, carry):
        for d in range(D // 16):
            rows2[r, pl.ds(d * 16, 16)] = zv
        return carry

    lax.fori_loop(0, C, zrow, 0, unroll=4)
    row0 = s * ROWS_PER_TILE
    rem = ROWS_PER_TILE % C
    zdescs = [
        pltpu.make_async_copy(rows2, acc.at[pl.ds(row0 + j * C, C)], zsem)
        for j in range(ROWS_PER_TILE // C)
    ] + [
        pltpu.make_async_copy(rows2.at[pl.ds(0, rem)],
                              acc.at[pl.ds(row0 + ROWS_PER_TILE - rem, rem)],
                              zsem)
    ]
    for zd in zdescs:
        zd.start()

    @pl.when(s == NS - 1)
    def _zero_tail():
        pltpu.sync_copy(rows2.at[pl.ds(0, TAIL)], acc.at[pl.ds(TAIL0, TAIL)])

    for t in (0, 1, 2):
        for d in idx_descs(t, t):
            d.start()
    for d in idx_descs(0, 0):
        d.wait()
    gather_desc(0).start()
    for d in idx_descs(1, 1):
        d.wait()
    gather_desc(1).start()

    for zd in zdescs:
        zd.wait()
    plsc.subcore_barrier()

    def step(i, carry):
        t0 = i * 3
        for b in (0, 1, 2):
            t = t0 + b
            nb = (b + 1) % 3
            pb = (b + 2) % 3

            gather_desc(b).wait()
            _scale_chunk(rows[b], w_v, b)
            snapshot_dst(b)

            scatter_start(b)

            @pl.when(t >= 1)
            def _drain():
                scatter_wait(pb)

            @pl.when(t + 2 < NCH)
            def _gather_ahead():
                for d in idx_descs(t + 2, pb):
                    d.wait()
                gather_desc(pb).start()

            @pl.when(t + 3 < NCH)
            def _prefetch():
                for d in idx_descs(t + 3, b):
                    d.start()
        return carry

    lax.fori_loop(0, NCH // 3, step, 0, unroll=False)

    scatter_wait((NCH - 1) % 3)

    plsc.subcore_barrier()
    pltpu.sync_copy(acc.at[pl.ds(row0, ROWS_PER_TILE)],
                    out_hbm.at[c, pl.ds(row0, ROWS_PER_TILE)])

    @pl.when(s == NS - 1)
    def _read_tail():
        pltpu.sync_copy(acc.at[pl.ds(TAIL0, TAIL)],
                        out_hbm.at[c, pl.ds(TAIL0, TAIL)])


def _sc_spmm(support, src_p, dst_p, w_p):
    mesh = plsc.VectorSubcoreMesh(core_axis_name="c", subcore_axis_name="s")
    f = pl.kernel(
        _sc_body,
        out_type=jax.ShapeDtypeStruct((NC, N, D), jnp.float32),
        mesh=mesh,
        scratch_types=[
            pltpu.VMEM((3, 3, C), jnp.int32),
            pltpu.VMEM((3, C), jnp.float32),
            pltpu.VMEM((C, D), jnp.float32),
            pltpu.VMEM((C, D), jnp.float32),
            pltpu.VMEM((C, D), jnp.float32),
            pltpu.VMEM_SHARED((N, D), jnp.float32),
        ] + [pltpu.SemaphoreType.DMA] * 16,
    )
    return f(support, src_p, dst_p, w_p)


def kernel(input, edge_index, edge_weight, W):
    src = edge_index[0].astype(jnp.int32)
    dst = edge_index[1].astype(jnp.int32)
    pad = EPAD - E
    spread = (jnp.arange(pad, dtype=jnp.int32) * 13) % N
    src_p = jnp.concatenate([src, spread])
    dst_p = jnp.concatenate([dst, spread])
    w_p = jnp.pad(edge_weight, (0, pad))
    support = _matmul(input, W)
    partials = _sc_spmm(support, src_p, dst_p, w_p)
    return _combine(partials)
